# baseline (speedup 1.0000x reference)
import functools

import jax
import jax.numpy as jnp
from jax import lax
from jax.experimental import pallas as pl
from jax.experimental.pallas import tpu as pltpu
from jax.experimental.pallas import tpu_sc as plsc

N = 10000
E = 320000
D_IN = 128
D_HID = 256
N_GP = 128
N_ADDON = 16
D_LAT = N_GP + N_ADDON
N_OUT = 256

NC = 2
NS = 16
L = 16

NPAD = 10240
EPAD = 327680
ROWS_PER_TILE = NPAD // NS
EROWS = EPAD // 128

f32 = jnp.float32
i32 = jnp.int32

_MESH = plsc.VectorSubcoreMesh(
    core_axis_name="c", subcore_axis_name="s", num_cores=NC, num_subcores=NS)
_SC_PARAMS = pltpu.CompilerParams(needs_layout_passes=False)
_SC_PARAMS_NOTC = pltpu.CompilerParams(
    needs_layout_passes=False, use_tc_tiling_on_sc=False)


def _fill_1d(ref, n, value):
    def body(i, _):
        ref[pl.ds(i * L, L)] = jnp.full((L,), value, ref.dtype)
        return 0
    lax.fori_loop(0, n // L, body, 0)


def _zero_rows(ref, nrows, ncols):
    z = jnp.zeros((L,), f32)
    def body(r, _):
        for j in range(ncols // L):
            ref[r, pl.ds(j * L, L)] = z
        return 0
    lax.fori_loop(0, nrows, body, 0)


_DEG_CH_ROWS = 8
_DEG_STEPS = (EROWS // NS) // _DEG_CH_ROWS


@functools.partial(
    pl.kernel,
    out_type=(jax.ShapeDtypeStruct((NPAD, 128), f32),
              jax.ShapeDtypeStruct((NPAD, 128), f32)),
    mesh=_MESH,
    compiler_params=_SC_PARAMS,
    scratch_types=[
        pltpu.VMEM_SHARED((NPAD,), f32),
        pltpu.VMEM((128,), f32),
        pltpu.VMEM((_DEG_CH_ROWS, 128), i32),
        pltpu.VMEM((ROWS_PER_TILE,), f32),
        pltpu.VMEM((320, 128), f32),
    ],
)
def _deg_kernel(dst2d, deg0_out, deg1_out, deg_sh, ones_v, didx_v, val_v, rep_v):
    cid = lax.axis_index("c")
    sid = lax.axis_index("s")

    _fill_1d(ones_v, 128, 1.0)
    _fill_1d(val_v, ROWS_PER_TILE, 0.0)

    @pl.when(cid == 0)
    def _():
        _fill_1d(val_v, ROWS_PER_TILE, 1.0)
    pltpu.sync_copy(val_v, deg_sh.at[pl.ds(sid * ROWS_PER_TILE, ROWS_PER_TILE)])
    plsc.subcore_barrier()

    rbase = cid * (EROWS // 2) + sid * (EROWS // (2 * NS))
    def step(ci, _):
        rb = rbase + ci * _DEG_CH_ROWS
        pltpu.sync_copy(dst2d.at[pl.ds(rb, _DEG_CH_ROWS)], didx_v)
        for j in range(_DEG_CH_ROWS):
            pltpu.sync_copy(ones_v, deg_sh.at[didx_v.at[j]], add=True)
        return 0
    lax.fori_loop(0, _DEG_STEPS // 2, step, 0)
    plsc.subcore_barrier()

    r0 = sid * ROWS_PER_TILE
    pltpu.sync_copy(deg_sh.at[pl.ds(r0, ROWS_PER_TILE)], val_v)
    for half in range(2):
        def rep_row(r, _):
            idx = jnp.full((L,), half * 320 + r, i32)
            row = plsc.load_gather(val_v, [idx])
            for j in range(128 // L):
                rep_v[r, pl.ds(j * L, L)] = row
            return 0
        lax.fori_loop(0, 320, rep_row, 0)
        @pl.when(cid == 0)
        def _():
            pltpu.sync_copy(rep_v, deg0_out.at[pl.ds(r0 + half * 320, 320)])
        @pl.when(cid == 1)
        def _():
            pltpu.sync_copy(rep_v, deg1_out.at[pl.ds(r0 + half * 320, 320)])


def _make_prop(edge_split):
    erows_core = (EROWS // NC) if edge_split else EROWS
    rows_tile = erows_core // NS
    n_ch = rows_tile
    G = 16 if edge_split else 32
    n_groups = n_ch // G

    @functools.partial(
        pl.kernel,
        out_type=(jax.ShapeDtypeStruct((NPAD, 128), f32),
                  jax.ShapeDtypeStruct((NPAD, 128), f32)),
        mesh=_MESH,
        compiler_params=_SC_PARAMS,
        scratch_types=[
            pltpu.VMEM_SHARED((NPAD, 128), f32),
            pltpu.VMEM((2 * G, 128), i32),
            pltpu.VMEM((128, 128), f32),
            pltpu.VMEM((128, 128), f32),
            pltpu.SemaphoreType.DMA,
            pltpu.SemaphoreType.DMA,
            pltpu.SemaphoreType.DMA,
            pltpu.SemaphoreType.DMA,
        ],
    )
    def prop(tbl0, tbl1, src2d, dst2d, out0, out1,
             acc_sh, istage, rows0, rows1,
             gsem0, gsem1, ssem0, ssem1):
        cid = lax.axis_index("c")
        sid = lax.axis_index("s")
        bufs = ((rows0, gsem0, ssem0),
                (rows1, gsem1, ssem1))

        _zero_rows(rows0, 128, 128)
        a0 = sid * ROWS_PER_TILE
        for z in range(ROWS_PER_TILE // 128):
            pltpu.sync_copy(rows0, acc_sh.at[pl.ds(a0 + z * 128, 128)])
        plsc.subcore_barrier()

        def run(tbl, rrow0):
            rbase = rrow0 + sid * rows_tile

            def fire_gather(b, l):
                rows, gsem, _ = bufs[b]
                pltpu.async_copy(tbl.at[istage.at[l]], rows, gsem)

            def wait_gather(b, l):
                rows, gsem, _ = bufs[b]
                pltpu.make_async_copy(tbl.at[istage.at[l]], rows,
                                      gsem).wait()

            def fire_scatter(b, l):
                rows, _, ssem = bufs[b]
                pltpu.async_copy(rows, acc_sh.at[istage.at[G + l]],
                                 ssem, add=True)

            def wait_scatter(b, l):
                rows, _, ssem = bufs[b]
                pltpu.make_async_copy(rows, acc_sh.at[istage.at[G + l]],
                                      ssem).wait()

            def group(gi, _):
                g0 = rbase + gi * G
                pltpu.sync_copy(src2d.at[pl.ds(g0, G)],
                                istage.at[pl.ds(0, G)])
                pltpu.sync_copy(dst2d.at[pl.ds(g0, G)],
                                istage.at[pl.ds(G, G)])
                fire_gather(0, 0)
                def pair(p, _):
                    l0 = 2 * p
                    @pl.when(p > 0)
                    def _():
                        wait_scatter(1, l0 - 1)
                    wait_gather(0, l0)
                    fire_scatter(0, l0)
                    fire_gather(1, l0 + 1)
                    wait_scatter(0, l0)
                    wait_gather(1, l0 + 1)
                    fire_scatter(1, l0 + 1)
                    @pl.when(p < G // 2 - 1)
                    def _():
                        fire_gather(0, l0 + 2)
                    return 0
                lax.fori_loop(0, G // 2, pair, 0)
                wait_scatter(1, G - 1)
                return 0
            lax.fori_loop(0, n_groups, group, 0)

        @pl.when(cid == 0)
        def _():
            run(tbl0, 0)
        @pl.when(cid == 1)
        def _():
            run(tbl1, erows_core if edge_split else 0)
        plsc.subcore_barrier()

        @pl.when(cid == 0)
        def _():
            pltpu.sync_copy(acc_sh.at[pl.ds(a0, ROWS_PER_TILE)],
                            out0.at[pl.ds(a0, ROWS_PER_TILE)])
        @pl.when(cid == 1)
        def _():
            pltpu.sync_copy(acc_sh.at[pl.ds(a0, ROWS_PER_TILE)],
                            out1.at[pl.ds(a0, ROWS_PER_TILE)])

    return prop


_prop_edge_split = _make_prop(True)
_prop_col_split = _make_prop(False)


_EDGE_ROWS_TILE = (EROWS // NC) // NS


_EDGE_NP2 = _EDGE_ROWS_TILE // 2


@functools.partial(
    pl.kernel,
    out_type=jax.ShapeDtypeStruct((EROWS, 128), f32),
    mesh=_MESH,
    compiler_params=_SC_PARAMS_NOTC,
    scratch_types=[
        pltpu.VMEM((32, 128), i32),
        pltpu.VMEM((128, 160), jnp.bfloat16),
        pltpu.VMEM((128, 160), jnp.bfloat16),
        pltpu.VMEM((128, 160), jnp.bfloat16),
        pltpu.VMEM((128, 160), jnp.bfloat16),
        pltpu.VMEM((1, 128), f32),
        pltpu.SemaphoreType.DMA,
        pltpu.SemaphoreType.DMA,
    ],
)
def _edge_kernel(zn, src2d, dst2d, out, istage, rs0, rs1, rd0, rd1, out_v,
                 sem0, sem1):
    cid = lax.axis_index("c")
    sid = lax.axis_index("s")
    wid = cid * NS + sid
    rbase = wid * _EDGE_ROWS_TILE
    iot = lax.iota(i32, L)
    bufs = ((rs0, rd0, sem0), (rs1, rd1, sem1))

    def load_fire(b, l):
        rs, rd, sem = bufs[b]
        pltpu.async_copy(zn.at[istage.at[l]], rs, sem)
        pltpu.async_copy(zn.at[istage.at[16 + l]], rd, sem)

    def wait_gathers(b, l):
        rs, rd, sem = bufs[b]
        pltpu.make_async_copy(zn.at[istage.at[l]], rs, sem).wait()
        pltpu.make_async_copy(zn.at[istage.at[16 + l]], rd, sem).wait()

    def compute(b, row):
        rs, rd, _ = bufs[b]

        @plsc.parallel_loop(0, 128, unroll=4)
        def _(r):
            acc = None
            for j in range(160 // 32):
                a = rs[r, pl.ds(j * 32, 32)]
                b_ = rd[r, pl.ds(j * 32, 32)]
                alo, ahi = plsc.unpack(a, format=plsc.PackFormat.INTERLEAVED)
                blo, bhi = plsc.unpack(b_, format=plsc.PackFormat.INTERLEAVED)
                t = alo * blo + ahi * bhi
                acc = t if acc is None else acc + t
            cum = plsc.cumsum(acc)
            plsc.store_scatter(out_v, [jnp.zeros((L,), i32),
                                       jnp.full((L,), r, i32)], cum,
                               mask=(iot == L - 1))

        pltpu.sync_copy(out_v, out.at[pl.ds(row, 1)])

    GE = 16
    def group(gi, _):
        g0 = rbase + gi * GE
        pltpu.sync_copy(src2d.at[pl.ds(g0, GE)], istage.at[pl.ds(0, GE)])
        pltpu.sync_copy(dst2d.at[pl.ds(g0, GE)], istage.at[pl.ds(GE, GE)])
        load_fire(0, 0)
        def pair(p, _):
            l0 = 2 * p
            load_fire(1, l0 + 1)
            wait_gathers(0, l0)
            compute(0, g0 + l0)
            @pl.when(p < GE // 2 - 1)
            def _():
                load_fire(0, l0 + 2)
            wait_gathers(1, l0 + 1)
            compute(1, g0 + l0 + 1)
            return 0
        lax.fori_loop(0, GE // 2, pair, 0)
        return 0
    lax.fori_loop(0, _EDGE_ROWS_TILE // GE, group, 0)


_BR = 2048
_GRID = NPAD // _BR


def _row_spec(w):
    return pl.BlockSpec((_BR, w), lambda i: (i, 0))


def _full_spec(a, b):
    return pl.BlockSpec((a, b), lambda i: (0, 0))


def _scale_body(x_ref, deg0_ref, deg1_ref, h_ref, hs_ref, isd_ref, ivd_ref):
    deg = deg0_ref[...] + deg1_ref[...]
    isd = lax.rsqrt(deg)
    h = jnp.log1p(x_ref[...])
    h_ref[...] = h
    hs_ref[...] = h * isd
    isd_ref[...] = isd
    ivd_ref[...] = 1.0 / deg


def _scale_kernel(xp, deg0_rep, deg1_rep):
    return pl.pallas_call(
        _scale_body,
        grid=(_GRID,),
        in_specs=[_row_spec(128)] * 3,
        out_specs=[_row_spec(128)] * 4,
        out_shape=[jax.ShapeDtypeStruct((NPAD, 128), f32)] * 4,
    )(xp, deg0_rep, deg1_rep)


def _mat1_body(a_ref, b_ref, h_ref, isd_ref, ivd_ref, w0_ref,
               h1_ref, hsa_ref, hsb_ref):
    isd = isd_ref[...]
    out1 = isd * (a_ref[...] + b_ref[...]) + ivd_ref[...] * h_ref[...]
    h1 = jnp.maximum(jnp.dot(out1, w0_ref[...],
                             preferred_element_type=f32), 0.0)
    h1_ref[...] = h1
    hsa_ref[...] = h1[:, :128] * isd
    hsb_ref[...] = h1[:, 128:] * isd


def _mat1_kernel(acc1a, acc1b, h, isd_rep, ivd_rep, W0):
    return pl.pallas_call(
        _mat1_body,
        grid=(_GRID,),
        in_specs=[_row_spec(128)] * 5 + [_full_spec(D_IN, D_HID)],
        out_specs=[_row_spec(256), _row_spec(128), _row_spec(128)],
        out_shape=[jax.ShapeDtypeStruct((NPAD, 256), f32),
                   jax.ShapeDtypeStruct((NPAD, 128), f32),
                   jax.ShapeDtypeStruct((NPAD, 128), f32)],
    )(acc1a, acc1b, h, isd_rep, ivd_rep, W0)


def _dec_body(a_ref, b_ref, h1_ref, isd_ref, ivd_ref, eps_ref,
              wmu_ref, wls_ref, wg_ref, wa_ref, mask_ref,
              mu_ref, ls_ref, zn_ref, gene_ref):
    isd = isd_ref[...]
    ivd = ivd_ref[...]
    h1 = h1_ref[...]
    pha = isd * a_ref[...] + ivd * h1[:, :128]
    phb = isd * b_ref[...] + ivd * h1[:, 128:]
    ph = jnp.concatenate([pha, phb], axis=1)
    mu = jnp.dot(ph, wmu_ref[...], preferred_element_type=f32)
    ls = jnp.clip(jnp.dot(ph, wls_ref[...], preferred_element_type=f32),
                  -5.0, 5.0)
    z = mu + jnp.exp(ls) * eps_ref[...]
    nrm = jnp.sqrt(jnp.sum(z * z, axis=1, keepdims=True))
    zn = z / (nrm + 1e-8)
    zn_ref[...] = jnp.concatenate(
        [zn, jnp.zeros((zn.shape[0], 160 - D_LAT), f32)],
        axis=1).astype(jnp.bfloat16)
    mu_ref[...] = mu
    ls_ref[...] = ls
    gene_ref[...] = (
        jnp.dot(z[:, :N_GP], wg_ref[...] * mask_ref[...],
                preferred_element_type=f32)
        + jnp.dot(z[:, N_GP:], wa_ref[...], preferred_element_type=f32))


def _dec_kernel(acc2a, acc2b, h1, isd_rep, ivd_rep, epsp,
                W_mu, W_logstd, W_gene, W_addon, mask):
    return pl.pallas_call(
        _dec_body,
        grid=(_GRID,),
        in_specs=[_row_spec(128), _row_spec(128), _row_spec(256),
                  _row_spec(128), _row_spec(128), _row_spec(D_LAT),
                  _full_spec(D_HID, D_LAT), _full_spec(D_HID, D_LAT),
                  _full_spec(N_GP, N_OUT), _full_spec(N_ADDON, N_OUT),
                  _full_spec(N_GP, N_OUT)],
        out_specs=[_row_spec(D_LAT), _row_spec(D_LAT), _row_spec(160),
                   _row_spec(N_OUT)],
        out_shape=[jax.ShapeDtypeStruct((NPAD, D_LAT), f32),
                   jax.ShapeDtypeStruct((NPAD, D_LAT), f32),
                   jax.ShapeDtypeStruct((NPAD, 160), jnp.bfloat16),
                   jax.ShapeDtypeStruct((NPAD, N_OUT), f32)],
    )(acc2a, acc2b, h1, isd_rep, ivd_rep, epsp,
      W_mu, W_logstd, W_gene, W_addon, mask)


def kernel(x, edge_index, eps, W0, W_mu, W_logstd, W_gene, W_addon, mask):
    src = edge_index[0]
    dst = edge_index[1]

    xp = jnp.pad(x, ((0, NPAD - N), (0, 0)))
    epsp = jnp.pad(eps, ((0, NPAD - N), (0, 0)))
    pad_ids = (N + (jnp.arange(EPAD - E) % (NPAD - N))).astype(i32)
    src2d = jnp.concatenate([src, pad_ids]).reshape(EROWS, 128)
    dst2d = jnp.concatenate([dst, pad_ids]).reshape(EROWS, 128)

    deg0_rep, deg1_rep = _deg_kernel(dst2d)
    h, hs, isd_rep, ivd_rep = _scale_kernel(xp, deg0_rep, deg1_rep)
    acc1a, acc1b = _prop_edge_split(hs, hs, src2d, dst2d)
    h1, hs1a, hs1b = _mat1_kernel(acc1a, acc1b, h, isd_rep, ivd_rep, W0)
    acc2a, acc2b = _prop_col_split(hs1a, hs1b, src2d, dst2d)
    mu, logstd, zn, gene = _dec_kernel(
        acc2a, acc2b, h1, isd_rep, ivd_rep, epsp,
        W_mu, W_logstd, W_gene, W_addon, mask)
    logits2d = _edge_kernel(zn, src2d, dst2d)

    return logits2d.reshape(EPAD)[:E], gene[:N], mu[:N], logstd[:N]


# speedup vs baseline: 1.0057x; 1.0057x over previous
import functools

import jax
import jax.numpy as jnp
from jax import lax
from jax.experimental import pallas as pl
from jax.experimental.pallas import tpu as pltpu
from jax.experimental.pallas import tpu_sc as plsc

N = 10000
E = 320000
D_IN = 128
D_HID = 256
N_GP = 128
N_ADDON = 16
D_LAT = N_GP + N_ADDON
N_OUT = 256

NC = 2
NS = 16
L = 16

NPAD = 10240
EPAD = 327680
ROWS_PER_TILE = NPAD // NS
EROWS = EPAD // 128

f32 = jnp.float32
i32 = jnp.int32

_MESH = plsc.VectorSubcoreMesh(
    core_axis_name="c", subcore_axis_name="s", num_cores=NC, num_subcores=NS)
_SC_PARAMS = pltpu.CompilerParams(needs_layout_passes=False)
_SC_PARAMS_NOTC = pltpu.CompilerParams(
    needs_layout_passes=False, use_tc_tiling_on_sc=False)


def _fill_1d(ref, n, value):
    def body(i, _):
        ref[pl.ds(i * L, L)] = jnp.full((L,), value, ref.dtype)
        return 0
    lax.fori_loop(0, n // L, body, 0)


def _zero_rows(ref, nrows, ncols):
    z = jnp.zeros((L,), f32)
    def body(r, _):
        for j in range(ncols // L):
            ref[r, pl.ds(j * L, L)] = z
        return 0
    lax.fori_loop(0, nrows, body, 0)


_DEG_CH_ROWS = 8
_DEG_STEPS = (EROWS // NS) // _DEG_CH_ROWS


@functools.partial(
    pl.kernel,
    out_type=(jax.ShapeDtypeStruct((NPAD, 128), f32),
              jax.ShapeDtypeStruct((NPAD, 128), f32)),
    mesh=_MESH,
    compiler_params=_SC_PARAMS,
    scratch_types=[
        pltpu.VMEM_SHARED((NPAD,), f32),
        pltpu.VMEM((128,), f32),
        pltpu.VMEM((2 * _DEG_CH_ROWS, 128), i32),
        pltpu.VMEM((ROWS_PER_TILE,), f32),
        pltpu.VMEM((320, 128), f32),
    ],
)
def _deg_kernel(sd2d, deg0_out, deg1_out, deg_sh, ones_v, didx_v, val_v, rep_v):
    cid = lax.axis_index("c")
    sid = lax.axis_index("s")

    _fill_1d(ones_v, 128, 1.0)
    _fill_1d(val_v, ROWS_PER_TILE, 0.0)

    @pl.when(cid == 0)
    def _():
        _fill_1d(val_v, ROWS_PER_TILE, 1.0)
    pltpu.sync_copy(val_v, deg_sh.at[pl.ds(sid * ROWS_PER_TILE, ROWS_PER_TILE)])
    plsc.subcore_barrier()

    rbase = cid * (EROWS // 2) + sid * (EROWS // (2 * NS))
    def step(ci, _):
        rb = 2 * (rbase + ci * _DEG_CH_ROWS)
        pltpu.sync_copy(sd2d.at[pl.ds(rb, 2 * _DEG_CH_ROWS)], didx_v)
        for j in range(_DEG_CH_ROWS):
            pltpu.sync_copy(ones_v, deg_sh.at[didx_v.at[2 * j + 1]], add=True)
        return 0
    lax.fori_loop(0, _DEG_STEPS // 2, step, 0)
    plsc.subcore_barrier()

    r0 = sid * ROWS_PER_TILE
    pltpu.sync_copy(deg_sh.at[pl.ds(r0, ROWS_PER_TILE)], val_v)
    for half in range(2):
        def rep_row(r, _):
            idx = jnp.full((L,), half * 320 + r, i32)
            row = plsc.load_gather(val_v, [idx])
            for j in range(128 // L):
                rep_v[r, pl.ds(j * L, L)] = row
            return 0
        lax.fori_loop(0, 320, rep_row, 0)
        @pl.when(cid == 0)
        def _():
            pltpu.sync_copy(rep_v, deg0_out.at[pl.ds(r0 + half * 320, 320)])
        @pl.when(cid == 1)
        def _():
            pltpu.sync_copy(rep_v, deg1_out.at[pl.ds(r0 + half * 320, 320)])


def _make_prop(edge_split):
    erows_core = (EROWS // NC) if edge_split else EROWS
    rows_tile = erows_core // NS
    n_ch = rows_tile
    G = 16 if edge_split else 32
    n_groups = n_ch // G

    @functools.partial(
        pl.kernel,
        out_type=(jax.ShapeDtypeStruct((NPAD, 128), f32),
                  jax.ShapeDtypeStruct((NPAD, 128), f32)),
        mesh=_MESH,
        compiler_params=_SC_PARAMS,
        scratch_types=[
            pltpu.VMEM_SHARED((NPAD, 128), f32),
            pltpu.VMEM((2 * G, 128), i32),
            pltpu.VMEM((128, 128), f32),
            pltpu.VMEM((128, 128), f32),
            pltpu.SemaphoreType.DMA,
            pltpu.SemaphoreType.DMA,
            pltpu.SemaphoreType.DMA,
            pltpu.SemaphoreType.DMA,
        ],
    )
    def prop(tbl0, tbl1, sd2d, out0, out1,
             acc_sh, istage, rows0, rows1,
             gsem0, gsem1, ssem0, ssem1):
        cid = lax.axis_index("c")
        sid = lax.axis_index("s")
        bufs = ((rows0, gsem0, ssem0),
                (rows1, gsem1, ssem1))

        _zero_rows(rows0, 128, 128)
        a0 = sid * ROWS_PER_TILE
        for z in range(ROWS_PER_TILE // 128):
            pltpu.sync_copy(rows0, acc_sh.at[pl.ds(a0 + z * 128, 128)])
        plsc.subcore_barrier()

        def run(tbl, rrow0):
            rbase = rrow0 + sid * rows_tile

            def fire_gather(b, l):
                rows, gsem, _ = bufs[b]
                pltpu.async_copy(tbl.at[istage.at[2 * l]], rows, gsem)

            def wait_gather(b, l):
                rows, gsem, _ = bufs[b]
                pltpu.make_async_copy(tbl.at[istage.at[2 * l]], rows,
                                      gsem).wait()

            def fire_scatter(b, l):
                rows, _, ssem = bufs[b]
                pltpu.async_copy(rows, acc_sh.at[istage.at[2 * l + 1]],
                                 ssem, add=True)

            def wait_scatter(b, l):
                rows, _, ssem = bufs[b]
                pltpu.make_async_copy(rows, acc_sh.at[istage.at[2 * l + 1]],
                                      ssem).wait()

            def group(gi, _):
                g0 = rbase + gi * G
                pltpu.sync_copy(sd2d.at[pl.ds(2 * g0, 2 * G)], istage)
                fire_gather(0, 0)
                def pair(p, _):
                    l0 = 2 * p
                    @pl.when(p > 0)
                    def _():
                        wait_scatter(1, l0 - 1)
                    wait_gather(0, l0)
                    fire_scatter(0, l0)
                    fire_gather(1, l0 + 1)
                    wait_scatter(0, l0)
                    wait_gather(1, l0 + 1)
                    fire_scatter(1, l0 + 1)
                    @pl.when(p < G // 2 - 1)
                    def _():
                        fire_gather(0, l0 + 2)
                    return 0
                lax.fori_loop(0, G // 2, pair, 0)
                wait_scatter(1, G - 1)
                return 0
            lax.fori_loop(0, n_groups, group, 0)

        @pl.when(cid == 0)
        def _():
            run(tbl0, 0)
        @pl.when(cid == 1)
        def _():
            run(tbl1, erows_core if edge_split else 0)
        plsc.subcore_barrier()

        @pl.when(cid == 0)
        def _():
            pltpu.sync_copy(acc_sh.at[pl.ds(a0, ROWS_PER_TILE)],
                            out0.at[pl.ds(a0, ROWS_PER_TILE)])
        @pl.when(cid == 1)
        def _():
            pltpu.sync_copy(acc_sh.at[pl.ds(a0, ROWS_PER_TILE)],
                            out1.at[pl.ds(a0, ROWS_PER_TILE)])

    return prop


_prop_edge_split = _make_prop(True)
_prop_col_split = _make_prop(False)


_EDGE_ROWS_TILE = (EROWS // NC) // NS


_EDGE_NP2 = _EDGE_ROWS_TILE // 2


@functools.partial(
    pl.kernel,
    out_type=jax.ShapeDtypeStruct((EROWS, 128), f32),
    mesh=_MESH,
    compiler_params=_SC_PARAMS_NOTC,
    scratch_types=[
        pltpu.VMEM((32, 128), i32),
        pltpu.VMEM((128, 160), jnp.bfloat16),
        pltpu.VMEM((128, 160), jnp.bfloat16),
        pltpu.VMEM((128, 160), jnp.bfloat16),
        pltpu.VMEM((128, 160), jnp.bfloat16),
        pltpu.VMEM((1, 128), f32),
        pltpu.SemaphoreType.DMA,
        pltpu.SemaphoreType.DMA,
    ],
)
def _edge_kernel(zn, sd2d, out, istage, rs0, rs1, rd0, rd1, out_v,
                 sem0, sem1):
    cid = lax.axis_index("c")
    sid = lax.axis_index("s")
    wid = cid * NS + sid
    rbase = wid * _EDGE_ROWS_TILE
    iot = lax.iota(i32, L)
    bufs = ((rs0, rd0, sem0), (rs1, rd1, sem1))

    def load_fire(b, l):
        rs, rd, sem = bufs[b]
        pltpu.async_copy(zn.at[istage.at[2 * l]], rs, sem)
        pltpu.async_copy(zn.at[istage.at[2 * l + 1]], rd, sem)

    def wait_gathers(b, l):
        rs, rd, sem = bufs[b]
        pltpu.make_async_copy(zn.at[istage.at[2 * l]], rs, sem).wait()
        pltpu.make_async_copy(zn.at[istage.at[2 * l + 1]], rd, sem).wait()

    def compute(b, row):
        rs, rd, _ = bufs[b]

        @plsc.parallel_loop(0, 128, unroll=4)
        def _(r):
            acc = None
            for j in range(160 // 32):
                a = rs[r, pl.ds(j * 32, 32)]
                b_ = rd[r, pl.ds(j * 32, 32)]
                alo, ahi = plsc.unpack(a, format=plsc.PackFormat.INTERLEAVED)
                blo, bhi = plsc.unpack(b_, format=plsc.PackFormat.INTERLEAVED)
                t = alo * blo + ahi * bhi
                acc = t if acc is None else acc + t
            cum = plsc.cumsum(acc)
            plsc.store_scatter(out_v, [jnp.zeros((L,), i32),
                                       jnp.full((L,), r, i32)], cum,
                               mask=(iot == L - 1))

        pltpu.sync_copy(out_v, out.at[pl.ds(row, 1)])

    GE = 16
    def group(gi, _):
        g0 = rbase + gi * GE
        pltpu.sync_copy(sd2d.at[pl.ds(2 * g0, 2 * GE)], istage)
        load_fire(0, 0)
        def pair(p, _):
            l0 = 2 * p
            load_fire(1, l0 + 1)
            wait_gathers(0, l0)
            compute(0, g0 + l0)
            @pl.when(p < GE // 2 - 1)
            def _():
                load_fire(0, l0 + 2)
            wait_gathers(1, l0 + 1)
            compute(1, g0 + l0 + 1)
            return 0
        lax.fori_loop(0, GE // 2, pair, 0)
        return 0
    lax.fori_loop(0, _EDGE_ROWS_TILE // GE, group, 0)


_BR = 2048
_GRID = NPAD // _BR


def _row_spec(w):
    return pl.BlockSpec((_BR, w), lambda i: (i, 0))


def _full_spec(a, b):
    return pl.BlockSpec((a, b), lambda i: (0, 0))


def _scale_body(x_ref, deg0_ref, deg1_ref, h_ref, hs_ref, isd_ref, ivd_ref):
    deg = deg0_ref[...] + deg1_ref[...]
    isd = lax.rsqrt(deg)
    h = jnp.log1p(x_ref[...])
    h_ref[...] = h
    hs_ref[...] = h * isd
    isd_ref[...] = isd
    ivd_ref[...] = 1.0 / deg


def _scale_kernel(xp, deg0_rep, deg1_rep):
    return pl.pallas_call(
        _scale_body,
        grid=(_GRID,),
        in_specs=[_row_spec(128)] * 3,
        out_specs=[_row_spec(128)] * 4,
        out_shape=[jax.ShapeDtypeStruct((NPAD, 128), f32)] * 4,
    )(xp, deg0_rep, deg1_rep)


def _mat1_body(a_ref, b_ref, h_ref, isd_ref, ivd_ref, w0_ref,
               h1_ref, hsa_ref, hsb_ref):
    isd = isd_ref[...]
    out1 = isd * (a_ref[...] + b_ref[...]) + ivd_ref[...] * h_ref[...]
    h1 = jnp.maximum(jnp.dot(out1, w0_ref[...],
                             preferred_element_type=f32), 0.0)
    h1_ref[...] = h1
    hsa_ref[...] = h1[:, :128] * isd
    hsb_ref[...] = h1[:, 128:] * isd


def _mat1_kernel(acc1a, acc1b, h, isd_rep, ivd_rep, W0):
    return pl.pallas_call(
        _mat1_body,
        grid=(_GRID,),
        in_specs=[_row_spec(128)] * 5 + [_full_spec(D_IN, D_HID)],
        out_specs=[_row_spec(256), _row_spec(128), _row_spec(128)],
        out_shape=[jax.ShapeDtypeStruct((NPAD, 256), f32),
                   jax.ShapeDtypeStruct((NPAD, 128), f32),
                   jax.ShapeDtypeStruct((NPAD, 128), f32)],
    )(acc1a, acc1b, h, isd_rep, ivd_rep, W0)


def _dec_body(a_ref, b_ref, h1_ref, isd_ref, ivd_ref, eps_ref,
              wmu_ref, wls_ref, wg_ref, wa_ref, mask_ref,
              mu_ref, ls_ref, zn_ref, gene_ref):
    isd = isd_ref[...]
    ivd = ivd_ref[...]
    h1 = h1_ref[...]
    pha = isd * a_ref[...] + ivd * h1[:, :128]
    phb = isd * b_ref[...] + ivd * h1[:, 128:]
    ph = jnp.concatenate([pha, phb], axis=1)
    mu = jnp.dot(ph, wmu_ref[...], preferred_element_type=f32)
    ls = jnp.clip(jnp.dot(ph, wls_ref[...], preferred_element_type=f32),
                  -5.0, 5.0)
    z = mu + jnp.exp(ls) * eps_ref[...]
    nrm = jnp.sqrt(jnp.sum(z * z, axis=1, keepdims=True))
    zn = z / (nrm + 1e-8)
    zn_ref[...] = jnp.concatenate(
        [zn, jnp.zeros((zn.shape[0], 160 - D_LAT), f32)],
        axis=1).astype(jnp.bfloat16)
    mu_ref[...] = mu
    ls_ref[...] = ls
    gene_ref[...] = (
        jnp.dot(z[:, :N_GP], wg_ref[...] * mask_ref[...],
                preferred_element_type=f32)
        + jnp.dot(z[:, N_GP:], wa_ref[...], preferred_element_type=f32))


def _dec_kernel(acc2a, acc2b, h1, isd_rep, ivd_rep, epsp,
                W_mu, W_logstd, W_gene, W_addon, mask):
    return pl.pallas_call(
        _dec_body,
        grid=(_GRID,),
        in_specs=[_row_spec(128), _row_spec(128), _row_spec(256),
                  _row_spec(128), _row_spec(128), _row_spec(D_LAT),
                  _full_spec(D_HID, D_LAT), _full_spec(D_HID, D_LAT),
                  _full_spec(N_GP, N_OUT), _full_spec(N_ADDON, N_OUT),
                  _full_spec(N_GP, N_OUT)],
        out_specs=[_row_spec(D_LAT), _row_spec(D_LAT), _row_spec(160),
                   _row_spec(N_OUT)],
        out_shape=[jax.ShapeDtypeStruct((NPAD, D_LAT), f32),
                   jax.ShapeDtypeStruct((NPAD, D_LAT), f32),
                   jax.ShapeDtypeStruct((NPAD, 160), jnp.bfloat16),
                   jax.ShapeDtypeStruct((NPAD, N_OUT), f32)],
    )(acc2a, acc2b, h1, isd_rep, ivd_rep, epsp,
      W_mu, W_logstd, W_gene, W_addon, mask)


def kernel(x, edge_index, eps, W0, W_mu, W_logstd, W_gene, W_addon, mask):
    src = edge_index[0]
    dst = edge_index[1]

    xp = jnp.pad(x, ((0, NPAD - N), (0, 0)))
    epsp = jnp.pad(eps, ((0, NPAD - N), (0, 0)))
    pad_ids = (N + (jnp.arange(EPAD - E) % (NPAD - N))).astype(i32)
    src2d = jnp.concatenate([src, pad_ids]).reshape(EROWS, 128)
    dst2d = jnp.concatenate([dst, pad_ids]).reshape(EROWS, 128)
    sd2d = jnp.stack([src2d, dst2d], axis=1).reshape(2 * EROWS, 128)
    del src2d, dst2d

    deg0_rep, deg1_rep = _deg_kernel(sd2d)
    h, hs, isd_rep, ivd_rep = _scale_kernel(xp, deg0_rep, deg1_rep)
    acc1a, acc1b = _prop_edge_split(hs, hs, sd2d)
    h1, hs1a, hs1b = _mat1_kernel(acc1a, acc1b, h, isd_rep, ivd_rep, W0)
    acc2a, acc2b = _prop_col_split(hs1a, hs1b, sd2d)
    mu, logstd, zn, gene = _dec_kernel(
        acc2a, acc2b, h1, isd_rep, ivd_rep, epsp,
        W_mu, W_logstd, W_gene, W_addon, mask)
    logits2d = _edge_kernel(zn, sd2d)

    return logits2d.reshape(EPAD)[:E], gene[:N], mu[:N], logstd[:N]
